# baseline (speedup 1.0000x reference)
import jax
import jax.numpy as jnp
from jax.experimental import pallas as pl
from jax.experimental.pallas import tpu as pltpu

_BB = 64


def _bin_embed_kernel(x_ref, low_ref, high_ref, w_ref, out_ref):
    K, D = w_ref.shape
    BB, F = x_ref.shape
    ii = jax.lax.broadcasted_iota(jnp.int32, (K, K), 0)
    kk = jax.lax.broadcasted_iota(jnp.int32, (K, K), 1)
    s = 1.0 / (jnp.abs(ii - kk) + 1).astype(jnp.float32)
    t = jnp.dot(s, w_ref[...], preferred_element_type=jnp.float32)

    x = x_ref[...].reshape(BB, 1, F)
    low = low_ref[...]
    high = high_ref[...]
    m = (x > low) & (x <= high)
    oh = jnp.where(m, 1.0, 0.0)
    oht = jnp.swapaxes(oh, 1, 2)
    ohf = oht.reshape(BB * F, K)
    o = jnp.dot(ohf, t, preferred_element_type=jnp.float32)
    out_ref[...] = o.reshape(BB, F, D)


def kernel(x, low, high, weight):
    B, F = x.shape
    K, D = weight.shape
    lowT = jnp.broadcast_to(low[:, None], (K, F)).reshape(1, K, F)
    highT = jnp.broadcast_to(high[:, None], (K, F)).reshape(1, K, F)

    return pl.pallas_call(
        _bin_embed_kernel,
        grid=(B // _BB,),
        in_specs=[
            pl.BlockSpec((_BB, F), lambda i: (i, 0)),
            pl.BlockSpec((1, K, F), lambda i: (0, 0, 0)),
            pl.BlockSpec((1, K, F), lambda i: (0, 0, 0)),
            pl.BlockSpec((K, D), lambda i: (0, 0)),
        ],
        out_specs=pl.BlockSpec((_BB, F, D), lambda i: (i, 0, 0)),
        out_shape=jax.ShapeDtypeStruct((B, F, D), jnp.float32),
        compiler_params=pltpu.CompilerParams(
            dimension_semantics=("parallel",),
        ),
    )(x, lowT, highT, weight)


# speedup vs baseline: 4.0634x; 4.0634x over previous
import jax
import jax.numpy as jnp
from jax.experimental import pallas as pl
from jax.experimental.pallas import tpu as pltpu

_NB = 128


def _bin_embed_kernel(xt_ref, low_ref, high_ref, w_ref, out_ref):
    K = w_ref.shape[0]
    F = xt_ref.shape[0]
    ii = jax.lax.broadcasted_iota(jnp.int32, (K, K), 0)
    kk = jax.lax.broadcasted_iota(jnp.int32, (K, K), 1)
    s = 1.0 / (jnp.abs(ii - kk) + 1).astype(jnp.float32)
    wt = jnp.swapaxes(w_ref[...], 0, 1)
    tt = jnp.dot(wt, s, preferred_element_type=jnp.float32)

    low = low_ref[...]
    high = high_ref[...]
    for f in range(F):
        xrow = xt_ref[f:f + 1, :]
        xb = jnp.broadcast_to(xrow, low.shape)
        m = (xb > low) & (xb <= high)
        oh = jnp.where(m, 1.0, 0.0)
        out_ref[f, :, :] = jnp.dot(tt, oh, preferred_element_type=jnp.float32)


def kernel(x, low, high, weight):
    B, F = x.shape
    K, D = weight.shape
    xt = x.T
    lowc = jnp.broadcast_to(low[:, None], (K, _NB))
    highc = jnp.broadcast_to(high[:, None], (K, _NB))

    out_t = pl.pallas_call(
        _bin_embed_kernel,
        grid=(B // _NB,),
        in_specs=[
            pl.BlockSpec((F, _NB), lambda i: (0, i)),
            pl.BlockSpec((K, _NB), lambda i: (0, 0)),
            pl.BlockSpec((K, _NB), lambda i: (0, 0)),
            pl.BlockSpec((K, D), lambda i: (0, 0)),
        ],
        out_specs=pl.BlockSpec((F, D, _NB), lambda i: (0, 0, i)),
        out_shape=jax.ShapeDtypeStruct((F, D, B), jnp.float32),
        compiler_params=pltpu.CompilerParams(
            dimension_semantics=("parallel",),
        ),
    )(xt, lowc, highc, weight)
    return jnp.transpose(out_t, (2, 0, 1))


# speedup vs baseline: 5.7706x; 1.4201x over previous
import jax
import jax.numpy as jnp
from jax.experimental import pallas as pl
from jax.experimental.pallas import tpu as pltpu

_NB = 256


def _bin_embed_kernel(xt_ref, low_ref, high_ref, w_ref, out_ref):
    K = w_ref.shape[0]
    F = xt_ref.shape[0]
    ii = jax.lax.broadcasted_iota(jnp.int32, (K, K), 0)
    kk = jax.lax.broadcasted_iota(jnp.int32, (K, K), 1)
    s = 1.0 / (jnp.abs(ii - kk) + 1).astype(jnp.float32)
    wt = jnp.swapaxes(w_ref[...], 0, 1)
    tt = jnp.dot(wt, s, preferred_element_type=jnp.float32)

    low = low_ref[...]
    high = high_ref[...]
    for f in range(F):
        xrow = xt_ref[f:f + 1, :]
        xb = jnp.broadcast_to(xrow, low.shape)
        m = (xb > low) & (xb <= high)
        oh = jnp.where(m, 1.0, 0.0)
        out_ref[f, :, :] = jnp.dot(tt, oh, preferred_element_type=jnp.float32)


def kernel(x, low, high, weight):
    B, F = x.shape
    K, D = weight.shape
    xt = x.T
    lowc = jnp.broadcast_to(low[:, None], (K, _NB))
    highc = jnp.broadcast_to(high[:, None], (K, _NB))

    out_t = pl.pallas_call(
        _bin_embed_kernel,
        grid=(B // _NB,),
        in_specs=[
            pl.BlockSpec((F, _NB), lambda i: (0, i)),
            pl.BlockSpec((K, _NB), lambda i: (0, 0)),
            pl.BlockSpec((K, _NB), lambda i: (0, 0)),
            pl.BlockSpec((K, D), lambda i: (0, 0)),
        ],
        out_specs=pl.BlockSpec((F, D, _NB), lambda i: (0, 0, i)),
        out_shape=jax.ShapeDtypeStruct((F, D, B), jnp.float32),
        compiler_params=pltpu.CompilerParams(
            dimension_semantics=("parallel",),
        ),
    )(xt, lowc, highc, weight)
    return jnp.transpose(out_t, (2, 0, 1))


# speedup vs baseline: 6.3852x; 1.1065x over previous
import jax
import jax.numpy as jnp
from jax.experimental import pallas as pl
from jax.experimental.pallas import tpu as pltpu

_NB = 512


def _bin_embed_kernel(xt_ref, low_ref, high_ref, w_ref, out_ref):
    K = w_ref.shape[0]
    F = xt_ref.shape[0]
    ii = jax.lax.broadcasted_iota(jnp.int32, (K, K), 0)
    kk = jax.lax.broadcasted_iota(jnp.int32, (K, K), 1)
    s = 1.0 / (jnp.abs(ii - kk) + 1).astype(jnp.float32)
    wt = jnp.swapaxes(w_ref[...], 0, 1)
    tt = jnp.dot(wt, s, preferred_element_type=jnp.float32)

    low = low_ref[...]
    high = high_ref[...]
    for f in range(F):
        xrow = xt_ref[f:f + 1, :]
        xb = jnp.broadcast_to(xrow, low.shape)
        m = (xb > low) & (xb <= high)
        oh = jnp.where(m, 1.0, 0.0)
        out_ref[f, :, :] = jnp.dot(tt, oh, preferred_element_type=jnp.float32)


def kernel(x, low, high, weight):
    B, F = x.shape
    K, D = weight.shape
    xt = x.T
    lowc = jnp.broadcast_to(low[:, None], (K, _NB))
    highc = jnp.broadcast_to(high[:, None], (K, _NB))

    out_t = pl.pallas_call(
        _bin_embed_kernel,
        grid=(B // _NB,),
        in_specs=[
            pl.BlockSpec((F, _NB), lambda i: (0, i)),
            pl.BlockSpec((K, _NB), lambda i: (0, 0)),
            pl.BlockSpec((K, _NB), lambda i: (0, 0)),
            pl.BlockSpec((K, D), lambda i: (0, 0)),
        ],
        out_specs=pl.BlockSpec((F, D, _NB), lambda i: (0, 0, i)),
        out_shape=jax.ShapeDtypeStruct((F, D, B), jnp.float32),
        compiler_params=pltpu.CompilerParams(
            dimension_semantics=("parallel",),
        ),
    )(xt, lowc, highc, weight)
    return jnp.transpose(out_t, (2, 0, 1))
